# baseline (speedup 1.0000x reference)
import functools

import jax
import jax.numpy as jnp
from jax import lax
from jax.experimental import pallas as pl
from jax.experimental.pallas import tpu as pltpu
from jax.experimental.pallas import tpu_sc as plsc

DIM = 64
DENSE = 128

NC = 2
NS = 16
NW = NC * NS
CHUNK = 100

RING = 8
AHEAD = 4


def _sc_gather_body(table_hbm, idx_hbm, out_hbm, idx_v, *rest):
    bufs = list(rest[:RING])
    gsem = list(rest[RING:2 * RING])
    wsem = list(rest[2 * RING:3 * RING])
    wid = lax.axis_index("s") * NC + lax.axis_index("c")
    nch = idx_hbm.shape[0] // NW
    base_row = wid * nch
    pltpu.sync_copy(idx_hbm.at[pl.ds(base_row, nch)], idx_v)

    def out_slice(g):
        return out_hbm.at[pl.ds((base_row + g) * CHUNK, CHUNK)]

    def gather(g, b):
        return pltpu.make_async_copy(table_hbm.at[idx_v.at[g]], bufs[b],
                                     gsem[b])

    def write(g, b):
        return pltpu.make_async_copy(bufs[b], out_slice(g), wsem[b])

    for j in range(AHEAD):
        gather(j, j).start()

    def body(i, carry):
        for j in range(RING):
            g = i * RING + j
            bn = (j + AHEAD) % RING

            @pl.when(g + AHEAD - RING >= 0)
            def _():
                write(g + AHEAD - RING, bn).wait()

            @pl.when(g + AHEAD < nch)
            def _():
                gather(g + AHEAD, bn).start()

            gather(g, j).wait()
            write(g, j).start()
        return carry

    lax.fori_loop(0, nch // RING, body, 0)
    for k in range(RING - AHEAD):
        g = nch - (RING - AHEAD) + k
        write(g, g % RING).wait()


def _tc_fold_body(t_ref, w_ref, b_ref, out_ref):
    acc = lax.dot_general(t_ref[...], w_ref[...], (((0,), (0,)), ((), ())),
                          preferred_element_type=jnp.float32)
    out_ref[...] = jnp.tanh(acc + b_ref[...])


def kernel(indices, table, W, b):
    B, L = indices.shape
    N = B * L
    V = table.shape[0]
    assert N % (NW * CHUNK) == 0 and (N // CHUNK // NW) % RING == 0
    idx2d = jnp.maximum(indices.reshape(N // CHUNK, CHUNK), 0)

    FBLK = 8192
    fold = pl.pallas_call(
        _tc_fold_body,
        grid=(pl.cdiv(V, FBLK),),
        in_specs=[
            pl.BlockSpec((DIM, FBLK), lambda i: (0, i)),
            pl.BlockSpec((DIM, DENSE), lambda i: (0, 0)),
            pl.BlockSpec((1, DENSE), lambda i: (0, 0)),
        ],
        out_specs=pl.BlockSpec((FBLK, DENSE), lambda i: (i, 0)),
        out_shape=jax.ShapeDtypeStruct((V, DENSE), jnp.float32),
    )(table.T, W, b.reshape(1, DENSE))

    mesh = plsc.VectorSubcoreMesh(core_axis_name="c", subcore_axis_name="s")
    gather = pl.kernel(
        _sc_gather_body,
        out_type=jax.ShapeDtypeStruct((N, DENSE), jnp.float32),
        mesh=mesh,
        scratch_types=(
            [pltpu.VMEM((N // CHUNK // NW, CHUNK), jnp.int32)]
            + [pltpu.VMEM((CHUNK, DENSE), jnp.float32)] * RING
            + [pltpu.SemaphoreType.DMA] * (2 * RING)
        ),
        compiler_params=pltpu.CompilerParams(use_tc_tiling_on_sc=False),
    )
    x = gather(fold, idx2d)
    return x.reshape(B, L, DENSE)


# speedup vs baseline: 1.0493x; 1.0493x over previous
import functools

import jax
import jax.numpy as jnp
from jax import lax
from jax.experimental import pallas as pl
from jax.experimental.pallas import tpu as pltpu
from jax.experimental.pallas import tpu_sc as plsc

DIM = 64
DENSE = 128

NC = 2
NS = 16
NW = NC * NS
CHUNK = 128

RING = 5
AHEAD = 3


def _sc_gather_body(table_hbm, idx_hbm, out_hbm, idx_v, *rest):
    bufs = list(rest[:RING])
    gsem = list(rest[RING:2 * RING])
    wsem = list(rest[2 * RING:3 * RING])
    wid = lax.axis_index("s") * NC + lax.axis_index("c")
    nch = idx_hbm.shape[0] // NW
    base_row = wid * nch
    pltpu.sync_copy(idx_hbm.at[pl.ds(base_row, nch)], idx_v)

    def out_slice(g):
        return out_hbm.at[pl.ds((base_row + g) * CHUNK, CHUNK)]

    def gather(g, b):
        return pltpu.make_async_copy(table_hbm.at[idx_v.at[g]], bufs[b],
                                     gsem[b])

    def write(g, b):
        return pltpu.make_async_copy(bufs[b], out_slice(g), wsem[b])

    for j in range(AHEAD):
        gather(j, j).start()

    def body(i, carry):
        for j in range(RING):
            g = i * RING + j
            bn = (j + AHEAD) % RING

            @pl.when(g + AHEAD - RING >= 0)
            def _():
                write(g + AHEAD - RING, bn).wait()

            @pl.when(g + AHEAD < nch)
            def _():
                gather(g + AHEAD, bn).start()

            gather(g, j).wait()
            write(g, j).start()
        return carry

    lax.fori_loop(0, nch // RING, body, 0)
    for k in range(RING - AHEAD):
        g = nch - (RING - AHEAD) + k
        write(g, g % RING).wait()


def _tc_fold_body(t_ref, w_ref, b_ref, out_ref):
    acc = lax.dot_general(t_ref[...], w_ref[...], (((0,), (0,)), ((), ())),
                          preferred_element_type=jnp.float32)
    out_ref[...] = jnp.tanh(acc + b_ref[...])


def kernel(indices, table, W, b):
    B, L = indices.shape
    N = B * L
    V = table.shape[0]
    assert N % (NW * CHUNK) == 0 and (N // CHUNK // NW) % RING == 0
    idx2d = jnp.maximum(indices.reshape(N // CHUNK, CHUNK), 0)

    FBLK = 16384
    fold = pl.pallas_call(
        _tc_fold_body,
        grid=(pl.cdiv(V, FBLK),),
        in_specs=[
            pl.BlockSpec((DIM, FBLK), lambda i: (0, i)),
            pl.BlockSpec((DIM, DENSE), lambda i: (0, 0)),
            pl.BlockSpec((1, DENSE), lambda i: (0, 0)),
        ],
        out_specs=pl.BlockSpec((FBLK, DENSE), lambda i: (i, 0)),
        out_shape=jax.ShapeDtypeStruct((V, DENSE), jnp.float32),
    )(table.T, W, b.reshape(1, DENSE))

    mesh = plsc.VectorSubcoreMesh(core_axis_name="c", subcore_axis_name="s")
    gather = pl.kernel(
        _sc_gather_body,
        out_type=jax.ShapeDtypeStruct((N, DENSE), jnp.float32),
        mesh=mesh,
        scratch_types=(
            [pltpu.VMEM((N // CHUNK // NW, CHUNK), jnp.int32)]
            + [pltpu.VMEM((CHUNK, DENSE), jnp.float32)] * RING
            + [pltpu.SemaphoreType.DMA] * (2 * RING)
        ),
        compiler_params=pltpu.CompilerParams(use_tc_tiling_on_sc=False),
    )
    x = gather(fold, idx2d)
    return x.reshape(B, L, DENSE)


# speedup vs baseline: 1.0637x; 1.0137x over previous
import functools

import jax
import jax.numpy as jnp
from jax import lax
from jax.experimental import pallas as pl
from jax.experimental.pallas import tpu as pltpu
from jax.experimental.pallas import tpu_sc as plsc

DIM = 64
DENSE = 128

NC = 2
NS = 16
NW = NC * NS
CHUNK = 128

RING = 5
AHEAD = 3


def _sc_gather_body(table_hbm, idx_hbm, out_hbm, idx_v, *rest):
    bufs = list(rest[:RING])
    gsem = list(rest[RING:2 * RING])
    wsem = list(rest[2 * RING:3 * RING])
    wid = lax.axis_index("s") * NC + lax.axis_index("c")
    nch = idx_hbm.shape[0] // NW
    base_row = wid * nch
    pltpu.sync_copy(idx_hbm.at[pl.ds(base_row, nch)], idx_v)

    def out_slice(g):
        return out_hbm.at[pl.ds((base_row + g) * CHUNK, CHUNK)]

    def gather(g, b):
        return pltpu.make_async_copy(table_hbm.at[idx_v.at[g]], bufs[b],
                                     gsem[b])

    def write(g, b):
        return pltpu.make_async_copy(bufs[b], out_slice(g), wsem[b])

    for j in range(AHEAD):
        gather(j, j).start()

    def body(i, carry):
        for j in range(RING):
            g = i * RING + j
            bn = (j + AHEAD) % RING

            @pl.when(g + AHEAD - RING >= 0)
            def _():
                write(g + AHEAD - RING, bn).wait()

            @pl.when(g + AHEAD < nch)
            def _():
                gather(g + AHEAD, bn).start()

            gather(g, j).wait()
            write(g, j).start()
        return carry

    lax.fori_loop(0, nch // RING, body, 0)
    for k in range(RING - AHEAD):
        g = nch - (RING - AHEAD) + k
        write(g, g % RING).wait()


def _tc_fold_body(t_ref, w_ref, b_ref, out_ref):
    acc = lax.dot_general(t_ref[...], w_ref[...], (((0,), (0,)), ((), ())),
                          preferred_element_type=jnp.float32)
    out_ref[...] = jnp.tanh(acc + b_ref[...])


def kernel(indices, table, W, b):
    B, L = indices.shape
    N = B * L
    V = table.shape[0]
    assert N % (NW * CHUNK) == 0 and (N // CHUNK // NW) % RING == 0
    idx2d = jnp.maximum(indices.reshape(N // CHUNK, CHUNK), 0)

    FBLK = 32768
    fold = pl.pallas_call(
        _tc_fold_body,
        grid=(pl.cdiv(V, FBLK),),
        in_specs=[
            pl.BlockSpec((DIM, FBLK), lambda i: (0, i)),
            pl.BlockSpec((DIM, DENSE), lambda i: (0, 0)),
            pl.BlockSpec((1, DENSE), lambda i: (0, 0)),
        ],
        out_specs=pl.BlockSpec((FBLK, DENSE), lambda i: (i, 0)),
        out_shape=jax.ShapeDtypeStruct((V, DENSE), jnp.float32),
    )(table.T, W, b.reshape(1, DENSE))

    mesh = plsc.VectorSubcoreMesh(core_axis_name="c", subcore_axis_name="s")
    gather = pl.kernel(
        _sc_gather_body,
        out_type=jax.ShapeDtypeStruct((N, DENSE), jnp.float32),
        mesh=mesh,
        scratch_types=(
            [pltpu.VMEM((N // CHUNK // NW, CHUNK), jnp.int32)]
            + [pltpu.VMEM((CHUNK, DENSE), jnp.float32)] * RING
            + [pltpu.SemaphoreType.DMA] * (2 * RING)
        ),
        compiler_params=pltpu.CompilerParams(use_tc_tiling_on_sc=False),
    )
    x = gather(fold, idx2d)
    return x.reshape(B, L, DENSE)
